# baseline (speedup 1.0000x reference)
import functools

import jax
import jax.numpy as jnp
from jax import lax
from jax.experimental import pallas as pl
from jax.experimental.pallas import tpu as pltpu
from jax.experimental.pallas import tpu_sc as plsc

N = 10000
E = 320000
NC = 2
NS = 16
NW = NC * NS
CH = 128
NCHUNK = 80
EPW = NCHUNK * CH
EPAD = NW * EPW
NPAD = 10240
RPW = NPAD // NS
DEGW = 16


@functools.cache
def _mesh():
    return plsc.VectorSubcoreMesh(
        core_axis_name="c", subcore_axis_name="s", num_cores=NC, num_subcores=NS)


@functools.cache
def _make_deg():
    @functools.partial(
        pl.kernel,
        out_type=jax.ShapeDtypeStruct((NC, NPAD, DEGW), jnp.float32),
        mesh=_mesh(),
        scratch_types=[
            pltpu.VMEM((NCHUNK, CH), jnp.int32),
            pltpu.VMEM((CH, DEGW), jnp.float32),
            pltpu.VMEM_SHARED((NPAD, DEGW), jnp.float32),
        ],
        compiler_params=pltpu.CompilerParams(use_tc_tiling_on_sc=False),
    )
    def _deg_kernel(dst_hbm, ones_hbm, zeros_hbm, out_hbm, dst_v, ones_v, acc):
        cid = lax.axis_index("c")
        sid = lax.axis_index("s")
        wid = sid * NC + cid
        rows = pl.ds(sid * RPW, RPW)
        pltpu.sync_copy(zeros_hbm.at[rows], acc.at[rows])
        pltpu.sync_copy(dst_hbm.at[wid], dst_v)
        pltpu.sync_copy(ones_hbm, ones_v)
        plsc.subcore_barrier()

        def body(j, carry):
            pltpu.sync_copy(ones_v, acc.at[dst_v.at[j]], add=True)
            return carry

        lax.fori_loop(0, NCHUNK, body, 0)
        plsc.subcore_barrier()
        pltpu.sync_copy(acc.at[rows], out_hbm.at[cid, rows])

    return _deg_kernel


@functools.cache
def _make_agg(d):
    @functools.partial(
        pl.kernel,
        out_type=jax.ShapeDtypeStruct((NC, NPAD, d), jnp.float32),
        mesh=_mesh(),
        scratch_types=[
            pltpu.VMEM((NCHUNK, CH), jnp.int32),
            pltpu.VMEM((NCHUNK, CH), jnp.int32),
            pltpu.VMEM((2, CH, d), jnp.float32),
            pltpu.VMEM_SHARED((NPAD, d), jnp.float32),
            pltpu.SemaphoreType.DMA,
            pltpu.SemaphoreType.DMA,
        ],
        compiler_params=pltpu.CompilerParams(use_tc_tiling_on_sc=False),
    )
    def k(m_hbm, src_hbm, dst_hbm, zeros_hbm, out_hbm, src_v, dst_v, gbuf, acc,
          sem_a, sem_b):
        cid = lax.axis_index("c")
        sid = lax.axis_index("s")
        wid = sid * NC + cid
        rows = pl.ds(sid * RPW, RPW)
        pltpu.sync_copy(zeros_hbm.at[rows], acc.at[rows])
        pltpu.sync_copy(src_hbm.at[wid], src_v)
        pltpu.sync_copy(dst_hbm.at[wid], dst_v)
        plsc.subcore_barrier()

        pltpu.async_copy(m_hbm.at[src_v.at[0]], gbuf.at[0], sem_a)

        def body(g, carry):
            j0 = 2 * g
            pltpu.async_copy(m_hbm.at[src_v.at[j0 + 1]], gbuf.at[1], sem_b)
            pltpu.make_async_copy(m_hbm.at[src_v.at[j0]], gbuf.at[0], sem_a).wait()
            pltpu.sync_copy(gbuf.at[0], acc.at[dst_v.at[j0]], add=True)

            @pl.when(j0 + 2 < NCHUNK)
            def _():
                pltpu.async_copy(m_hbm.at[src_v.at[j0 + 2]], gbuf.at[0], sem_a)

            pltpu.make_async_copy(m_hbm.at[src_v.at[j0 + 1]], gbuf.at[1],
                                  sem_b).wait()
            pltpu.sync_copy(gbuf.at[1], acc.at[dst_v.at[j0 + 1]], add=True)
            return carry

        lax.fori_loop(0, NCHUNK // 2, body, 0)
        plsc.subcore_barrier()
        pltpu.sync_copy(acc.at[rows], out_hbm.at[cid, rows])

    return k


def _pre_body(x_ref, w_ref, degp_ref, m_ref, dinv_ref):
    deg = degp_ref[0][0:N, 0:1] + degp_ref[1][0:N, 0:1] + 1.0
    dinv = lax.rsqrt(deg)
    dinv_ref[...] = dinv
    m_ref[...] = jnp.dot(x_ref[...], w_ref[...],
                         preferred_element_type=jnp.float32) * dinv


_pre = pl.pallas_call(
    _pre_body,
    out_shape=[jax.ShapeDtypeStruct((N, 64), jnp.float32),
               jax.ShapeDtypeStruct((N, 1), jnp.float32)],
)


def _mid_body(a_ref, m_ref, dinv_ref, b_ref, g_ref, bt_ref, w_ref, o_ref):
    dinv = dinv_ref[...]
    t = (a_ref[0][0:N] + a_ref[1][0:N] + m_ref[...]) * dinv + b_ref[...]
    mu = jnp.mean(t, axis=0, keepdims=True)
    var = jnp.mean(jnp.square(t - mu), axis=0, keepdims=True)
    t = (t - mu) * lax.rsqrt(var + 1e-5) * g_ref[...] + bt_ref[...]
    t = jnp.maximum(t, 0.0)
    o_ref[...] = jnp.dot(t, w_ref[...], preferred_element_type=jnp.float32) * dinv


def _mid_nomat_body(a_ref, m_ref, dinv_ref, b_ref, g_ref, bt_ref, o_ref):
    dinv = dinv_ref[...]
    t = (a_ref[0][0:N] + a_ref[1][0:N] + m_ref[...]) * dinv + b_ref[...]
    mu = jnp.mean(t, axis=0, keepdims=True)
    var = jnp.mean(jnp.square(t - mu), axis=0, keepdims=True)
    t = (t - mu) * lax.rsqrt(var + 1e-5) * g_ref[...] + bt_ref[...]
    t = jnp.maximum(t, 0.0)
    o_ref[...] = t * dinv


def _fin_body(a_ref, m_ref, dinv_ref, w_ref, b_ref, o_ref):
    t = (a_ref[0][0:N] + a_ref[1][0:N] + m_ref[...]) * dinv_ref[...]
    h = jnp.dot(t, w_ref[...], preferred_element_type=jnp.float32) + b_ref[...]
    mx = jnp.max(h, axis=1, keepdims=True)
    lse = jnp.log(jnp.sum(jnp.exp(h - mx), axis=1, keepdims=True)) + mx
    o_ref[...] = h - lse


def _make_mid(dout):
    return pl.pallas_call(
        _mid_body, out_shape=jax.ShapeDtypeStruct((N, dout), jnp.float32))


_mid12 = _make_mid(32)
_mid23 = _make_mid(16)
_mid34 = pl.pallas_call(
    _mid_nomat_body, out_shape=jax.ShapeDtypeStruct((N, 16), jnp.float32))
_fin = pl.pallas_call(
    _fin_body, out_shape=jax.ShapeDtypeStruct((N, 40), jnp.float32))


def kernel(x, W1, b1, g1, bt1, W2, b2, g2, bt2, W3, b3, g3, bt3, W4, b4,
           edge_index):
    pad = EPAD - E
    src = jnp.concatenate(
        [edge_index[0], jnp.zeros((pad,), jnp.int32)]).reshape(NW, NCHUNK, CH)
    dst = jnp.concatenate(
        [edge_index[1], jnp.full((pad,), N, jnp.int32)]).reshape(NW, NCHUNK, CH)
    ones = jnp.ones((CH, DEGW), jnp.float32)
    z_deg = jnp.zeros((NPAD, DEGW), jnp.float32)
    z64 = jnp.zeros((NPAD, 64), jnp.float32)
    z32 = jnp.zeros((NPAD, 32), jnp.float32)
    z16 = jnp.zeros((NPAD, 16), jnp.float32)

    degp = _make_deg()(dst, ones, z_deg)
    m1, dinv = _pre(x, W1, degp)
    a1 = _make_agg(64)(m1, src, dst, z64)
    m2 = _mid12(a1, m1, dinv, b1.reshape(1, -1), g1.reshape(1, -1),
                bt1.reshape(1, -1), W2)
    a2 = _make_agg(32)(m2, src, dst, z32)
    m3 = _mid23(a2, m2, dinv, b2.reshape(1, -1), g2.reshape(1, -1),
                bt2.reshape(1, -1), W3)
    a3 = _make_agg(16)(m3, src, dst, z16)
    m4 = _mid34(a3, m3, dinv, b3.reshape(1, -1), g3.reshape(1, -1),
                bt3.reshape(1, -1))
    a4 = _make_agg(16)(m4, src, dst, z16)
    return _fin(a4, m4, dinv, W4, b4.reshape(1, -1))


# speedup vs baseline: 1.0487x; 1.0487x over previous
import functools

import jax
import jax.numpy as jnp
from jax import lax
from jax.experimental import pallas as pl
from jax.experimental.pallas import tpu as pltpu
from jax.experimental.pallas import tpu_sc as plsc

N = 10000
E = 320000
NC = 2
NS = 16
NW = NC * NS
CH = 128
NCHUNK = 80
EPW = NCHUNK * CH
EPAD = NW * EPW
NPAD = 10240
RPW = NPAD // NS
DEGW = 16


@functools.cache
def _mesh():
    return plsc.VectorSubcoreMesh(
        core_axis_name="c", subcore_axis_name="s", num_cores=NC, num_subcores=NS)


@functools.cache
def _make_deg():
    @functools.partial(
        pl.kernel,
        out_type=jax.ShapeDtypeStruct((NC, NPAD, DEGW), jnp.float32),
        mesh=_mesh(),
        scratch_types=[
            pltpu.VMEM((NCHUNK, CH), jnp.int32),
            pltpu.VMEM((CH, DEGW), jnp.float32),
            pltpu.VMEM_SHARED((NPAD, DEGW), jnp.float32),
        ],
        compiler_params=pltpu.CompilerParams(use_tc_tiling_on_sc=False),
    )
    def _deg_kernel(dst_hbm, ones_hbm, zeros_hbm, out_hbm, dst_v, ones_v, acc):
        cid = lax.axis_index("c")
        sid = lax.axis_index("s")
        wid = sid * NC + cid
        rows = pl.ds(sid * RPW, RPW)
        pltpu.sync_copy(zeros_hbm.at[rows], acc.at[rows])
        pltpu.sync_copy(dst_hbm.at[wid], dst_v)
        pltpu.sync_copy(ones_hbm, ones_v)
        plsc.subcore_barrier()

        def body(j, carry):
            pltpu.sync_copy(ones_v, acc.at[dst_v.at[j]], add=True)
            return carry

        lax.fori_loop(0, NCHUNK, body, 0)
        plsc.subcore_barrier()
        pltpu.sync_copy(acc.at[rows], out_hbm.at[cid, rows])

    return _deg_kernel


@functools.cache
def _make_agg(d):
    @functools.partial(
        pl.kernel,
        out_type=jax.ShapeDtypeStruct((NC, NPAD, d), jnp.float32),
        mesh=_mesh(),
        scratch_types=[
            pltpu.VMEM((NCHUNK, CH), jnp.int32),
            pltpu.VMEM((NCHUNK, CH), jnp.int32),
            pltpu.VMEM((2, CH, d), jnp.float32),
            pltpu.VMEM_SHARED((NPAD, d), jnp.float32),
            pltpu.SemaphoreType.DMA,
            pltpu.SemaphoreType.DMA,
        ],
        compiler_params=pltpu.CompilerParams(use_tc_tiling_on_sc=False),
    )
    def k(m_hbm, src_hbm, dst_hbm, zeros_hbm, out_hbm, src_v, dst_v, gbuf, acc,
          sem_a, sem_b):
        cid = lax.axis_index("c")
        sid = lax.axis_index("s")
        wid = sid * NC + cid
        rows = pl.ds(sid * RPW, RPW)
        pltpu.sync_copy(zeros_hbm.at[rows], acc.at[rows])
        pltpu.sync_copy(src_hbm.at[wid], src_v)
        pltpu.sync_copy(dst_hbm.at[wid], dst_v)
        plsc.subcore_barrier()

        pltpu.async_copy(m_hbm.at[src_v.at[0]], gbuf.at[0], sem_a)

        def body(g, carry):
            j0 = 2 * g
            pltpu.async_copy(m_hbm.at[src_v.at[j0 + 1]], gbuf.at[1], sem_b)
            pltpu.make_async_copy(m_hbm.at[src_v.at[j0]], gbuf.at[0], sem_a).wait()
            pltpu.sync_copy(gbuf.at[0], acc.at[dst_v.at[j0]], add=True)

            @pl.when(j0 + 2 < NCHUNK)
            def _():
                pltpu.async_copy(m_hbm.at[src_v.at[j0 + 2]], gbuf.at[0], sem_a)

            pltpu.make_async_copy(m_hbm.at[src_v.at[j0 + 1]], gbuf.at[1],
                                  sem_b).wait()
            pltpu.sync_copy(gbuf.at[1], acc.at[dst_v.at[j0 + 1]], add=True)
            return carry

        lax.fori_loop(0, NCHUNK // 2, body, 0)
        plsc.subcore_barrier()
        pltpu.sync_copy(acc.at[rows], out_hbm.at[cid, rows])

    return k


def _pre_body(x_ref, w_ref, degp_ref, m_ref, dinv_ref):
    deg = degp_ref[0][0:N, 0:1] + degp_ref[1][0:N, 0:1] + 1.0
    dinv = lax.rsqrt(deg)
    dinv_ref[...] = dinv
    m_ref[...] = jnp.dot(x_ref[...], w_ref[...],
                         preferred_element_type=jnp.float32) * dinv


_pre = pl.pallas_call(
    _pre_body,
    out_shape=[jax.ShapeDtypeStruct((N, 64), jnp.float32),
               jax.ShapeDtypeStruct((N, 1), jnp.float32)],
)


def _mid_body(a_ref, m_ref, dinv_ref, b_ref, g_ref, bt_ref, w_ref, o_ref):
    dinv = dinv_ref[...]
    t = (a_ref[0][0:N] + a_ref[1][0:N] + m_ref[...]) * dinv + b_ref[...]
    mu = jnp.mean(t, axis=0, keepdims=True)
    var = jnp.mean(jnp.square(t - mu), axis=0, keepdims=True)
    t = (t - mu) * lax.rsqrt(var + 1e-5) * g_ref[...] + bt_ref[...]
    t = jnp.maximum(t, 0.0)
    o_ref[...] = jnp.dot(t, w_ref[...], preferred_element_type=jnp.float32) * dinv


def _mid_nomat_body(a_ref, m_ref, dinv_ref, b_ref, g_ref, bt_ref, o_ref):
    dinv = dinv_ref[...]
    t = (a_ref[0][0:N] + a_ref[1][0:N] + m_ref[...]) * dinv + b_ref[...]
    mu = jnp.mean(t, axis=0, keepdims=True)
    var = jnp.mean(jnp.square(t - mu), axis=0, keepdims=True)
    t = (t - mu) * lax.rsqrt(var + 1e-5) * g_ref[...] + bt_ref[...]
    t = jnp.maximum(t, 0.0)
    o_ref[...] = t * dinv


def _fin_body(a_ref, m_ref, dinv_ref, w_ref, b_ref, o_ref):
    t = (a_ref[0][0:N] + a_ref[1][0:N] + m_ref[...]) * dinv_ref[...]
    h = jnp.dot(t, w_ref[...], preferred_element_type=jnp.float32) + b_ref[...]
    mx = jnp.max(h, axis=1, keepdims=True)
    lse = jnp.log(jnp.sum(jnp.exp(h - mx), axis=1, keepdims=True)) + mx
    o_ref[...] = h - lse


def _make_mid(dout):
    return pl.pallas_call(
        _mid_body, out_shape=jax.ShapeDtypeStruct((N, dout), jnp.float32))


_mid12 = _make_mid(32)
_mid23 = _make_mid(16)
_mid34 = pl.pallas_call(
    _mid_nomat_body, out_shape=jax.ShapeDtypeStruct((N, 16), jnp.float32))
_fin = pl.pallas_call(
    _fin_body, out_shape=jax.ShapeDtypeStruct((N, 40), jnp.float32))


def kernel(x, W1, b1, g1, bt1, W2, b2, g2, bt2, W3, b3, g3, bt3, W4, b4,
           edge_index):
    ppw = EPW - E // NW
    src = jnp.concatenate(
        [edge_index[0].reshape(NW, E // NW),
         jnp.zeros((NW, ppw), jnp.int32)], axis=1).reshape(NW, NCHUNK, CH)
    trash = jnp.broadcast_to(N + jnp.arange(ppw, dtype=jnp.int32), (NW, ppw))
    dst = jnp.concatenate(
        [edge_index[1].reshape(NW, E // NW), trash],
        axis=1).reshape(NW, NCHUNK, CH)
    ones = jnp.ones((CH, DEGW), jnp.float32)
    z_deg = jnp.zeros((NPAD, DEGW), jnp.float32)
    z64 = jnp.zeros((NPAD, 64), jnp.float32)
    z32 = jnp.zeros((NPAD, 32), jnp.float32)
    z16 = jnp.zeros((NPAD, 16), jnp.float32)

    degp = _make_deg()(dst, ones, z_deg)
    m1, dinv = _pre(x, W1, degp)
    a1 = _make_agg(64)(m1, src, dst, z64)
    m2 = _mid12(a1, m1, dinv, b1.reshape(1, -1), g1.reshape(1, -1),
                bt1.reshape(1, -1), W2)
    a2 = _make_agg(32)(m2, src, dst, z32)
    m3 = _mid23(a2, m2, dinv, b2.reshape(1, -1), g2.reshape(1, -1),
                bt2.reshape(1, -1), W3)
    a3 = _make_agg(16)(m3, src, dst, z16)
    m4 = _mid34(a3, m3, dinv, b3.reshape(1, -1), g3.reshape(1, -1),
                bt3.reshape(1, -1))
    a4 = _make_agg(16)(m4, src, dst, z16)
    return _fin(a4, m4, dinv, W4, b4.reshape(1, -1))


# speedup vs baseline: 1.5809x; 1.5075x over previous
import functools

import jax
import jax.numpy as jnp
from jax import lax
from jax.experimental import pallas as pl
from jax.experimental.pallas import tpu as pltpu
from jax.experimental.pallas import tpu_sc as plsc

N = 10000
E = 320000
NC = 2
NS = 16
NW = NC * NS
CH = 80
EPW = E // NW
NCHUNK = EPW // CH
NPAD = 10240
RPW = NPAD // NS
DEGW = 16


@functools.cache
def _mesh():
    return plsc.VectorSubcoreMesh(
        core_axis_name="c", subcore_axis_name="s", num_cores=NC, num_subcores=NS)


@functools.cache
def _make_deg():
    @functools.partial(
        pl.kernel,
        out_type=jax.ShapeDtypeStruct((NC, NPAD, DEGW), jnp.float32),
        mesh=_mesh(),
        scratch_types=[
            pltpu.VMEM((NCHUNK, CH), jnp.int32),
            pltpu.VMEM((CH, DEGW), jnp.float32),
            pltpu.VMEM_SHARED((NPAD, DEGW), jnp.float32),
        ],
        compiler_params=pltpu.CompilerParams(use_tc_tiling_on_sc=False),
    )
    def _deg_kernel(dst_hbm, ones_hbm, zeros_hbm, out_hbm, dst_v, ones_v, acc):
        cid = lax.axis_index("c")
        sid = lax.axis_index("s")
        wid = sid * NC + cid
        rows = pl.ds(sid * RPW, RPW)
        pltpu.sync_copy(zeros_hbm.at[rows], acc.at[rows])
        pltpu.sync_copy(dst_hbm.at[wid], dst_v)
        pltpu.sync_copy(ones_hbm, ones_v)
        plsc.subcore_barrier()

        def body(j, carry):
            pltpu.sync_copy(ones_v, acc.at[dst_v.at[j]], add=True)
            return carry

        lax.fori_loop(0, NCHUNK, body, 0)
        plsc.subcore_barrier()
        pltpu.sync_copy(acc.at[rows], out_hbm.at[cid, rows])

    return _deg_kernel


@functools.cache
def _make_agg(d):
    @functools.partial(
        pl.kernel,
        out_type=jax.ShapeDtypeStruct((NC, NPAD, d), jnp.float32),
        mesh=_mesh(),
        scratch_types=[
            pltpu.VMEM((NCHUNK, CH), jnp.int32),
            pltpu.VMEM((NCHUNK, CH), jnp.int32),
            pltpu.VMEM((2, CH, d), jnp.float32),
            pltpu.VMEM_SHARED((NPAD, d), jnp.float32),
            pltpu.SemaphoreType.DMA,
            pltpu.SemaphoreType.DMA,
        ],
        compiler_params=pltpu.CompilerParams(use_tc_tiling_on_sc=False),
    )
    def k(m_hbm, src_hbm, dst_hbm, zeros_hbm, out_hbm, src_v, dst_v, gbuf, acc,
          sem_a, sem_b):
        cid = lax.axis_index("c")
        sid = lax.axis_index("s")
        wid = sid * NC + cid
        rows = pl.ds(sid * RPW, RPW)
        pltpu.sync_copy(zeros_hbm.at[rows], acc.at[rows])
        pltpu.sync_copy(src_hbm.at[wid], src_v)
        pltpu.sync_copy(dst_hbm.at[wid], dst_v)
        plsc.subcore_barrier()

        pltpu.async_copy(m_hbm.at[src_v.at[0]], gbuf.at[0], sem_a)

        def body(g, carry):
            j0 = 2 * g
            pltpu.async_copy(m_hbm.at[src_v.at[j0 + 1]], gbuf.at[1], sem_b)
            pltpu.make_async_copy(m_hbm.at[src_v.at[j0]], gbuf.at[0], sem_a).wait()
            pltpu.sync_copy(gbuf.at[0], acc.at[dst_v.at[j0]], add=True)

            @pl.when(j0 + 2 < NCHUNK)
            def _():
                pltpu.async_copy(m_hbm.at[src_v.at[j0 + 2]], gbuf.at[0], sem_a)

            pltpu.make_async_copy(m_hbm.at[src_v.at[j0 + 1]], gbuf.at[1],
                                  sem_b).wait()
            pltpu.sync_copy(gbuf.at[1], acc.at[dst_v.at[j0 + 1]], add=True)
            return carry

        lax.fori_loop(0, NCHUNK // 2, body, 0)
        pltpu.make_async_copy(m_hbm.at[src_v.at[NCHUNK - 1]], gbuf.at[0],
                              sem_a).wait()
        pltpu.sync_copy(gbuf.at[0], acc.at[dst_v.at[NCHUNK - 1]], add=True)
        plsc.subcore_barrier()
        pltpu.sync_copy(acc.at[rows], out_hbm.at[cid, rows])

    return k


def _pre_body(x_ref, w_ref, degp_ref, m_ref, dinv_ref):
    deg = degp_ref[0][0:N, 0:1] + degp_ref[1][0:N, 0:1] + 1.0
    dinv = lax.rsqrt(deg)
    dinv_ref[...] = dinv
    m_ref[...] = jnp.dot(x_ref[...], w_ref[...],
                         preferred_element_type=jnp.float32) * dinv


_pre = pl.pallas_call(
    _pre_body,
    out_shape=[jax.ShapeDtypeStruct((N, 64), jnp.float32),
               jax.ShapeDtypeStruct((N, 1), jnp.float32)],
)


def _mid_body(a_ref, m_ref, dinv_ref, b_ref, g_ref, bt_ref, w_ref, o_ref):
    dinv = dinv_ref[...]
    t = (a_ref[0][0:N] + a_ref[1][0:N] + m_ref[...]) * dinv + b_ref[...]
    mu = jnp.mean(t, axis=0, keepdims=True)
    var = jnp.mean(jnp.square(t - mu), axis=0, keepdims=True)
    t = (t - mu) * lax.rsqrt(var + 1e-5) * g_ref[...] + bt_ref[...]
    t = jnp.maximum(t, 0.0)
    o_ref[...] = jnp.dot(t, w_ref[...], preferred_element_type=jnp.float32) * dinv


def _mid_nomat_body(a_ref, m_ref, dinv_ref, b_ref, g_ref, bt_ref, o_ref):
    dinv = dinv_ref[...]
    t = (a_ref[0][0:N] + a_ref[1][0:N] + m_ref[...]) * dinv + b_ref[...]
    mu = jnp.mean(t, axis=0, keepdims=True)
    var = jnp.mean(jnp.square(t - mu), axis=0, keepdims=True)
    t = (t - mu) * lax.rsqrt(var + 1e-5) * g_ref[...] + bt_ref[...]
    t = jnp.maximum(t, 0.0)
    o_ref[...] = t * dinv


def _fin_body(a_ref, m_ref, dinv_ref, w_ref, b_ref, o_ref):
    t = (a_ref[0][0:N] + a_ref[1][0:N] + m_ref[...]) * dinv_ref[...]
    h = jnp.dot(t, w_ref[...], preferred_element_type=jnp.float32) + b_ref[...]
    mx = jnp.max(h, axis=1, keepdims=True)
    lse = jnp.log(jnp.sum(jnp.exp(h - mx), axis=1, keepdims=True)) + mx
    o_ref[...] = h - lse


def _make_mid(dout):
    return pl.pallas_call(
        _mid_body, out_shape=jax.ShapeDtypeStruct((N, dout), jnp.float32))


_mid12 = _make_mid(32)
_mid23 = _make_mid(16)
_mid34 = pl.pallas_call(
    _mid_nomat_body, out_shape=jax.ShapeDtypeStruct((N, 16), jnp.float32))
_fin = pl.pallas_call(
    _fin_body, out_shape=jax.ShapeDtypeStruct((N, 40), jnp.float32))


def kernel(x, W1, b1, g1, bt1, W2, b2, g2, bt2, W3, b3, g3, bt3, W4, b4,
           edge_index):
    src = edge_index[0].reshape(NW, NCHUNK, CH)
    dst = edge_index[1].reshape(NW, NCHUNK, CH)
    ones = jnp.ones((CH, DEGW), jnp.float32)
    z_deg = jnp.zeros((NPAD, DEGW), jnp.float32)
    z64 = jnp.zeros((NPAD, 64), jnp.float32)
    z32 = jnp.zeros((NPAD, 32), jnp.float32)
    z16 = jnp.zeros((NPAD, 16), jnp.float32)

    degp = _make_deg()(dst, ones, z_deg)
    m1, dinv = _pre(x, W1, degp)
    a1 = _make_agg(64)(m1, src, dst, z64)
    m2 = _mid12(a1, m1, dinv, b1.reshape(1, -1), g1.reshape(1, -1),
                bt1.reshape(1, -1), W2)
    a2 = _make_agg(32)(m2, src, dst, z32)
    m3 = _mid23(a2, m2, dinv, b2.reshape(1, -1), g2.reshape(1, -1),
                bt2.reshape(1, -1), W3)
    a3 = _make_agg(16)(m3, src, dst, z16)
    m4 = _mid34(a3, m3, dinv, b3.reshape(1, -1), g3.reshape(1, -1),
                bt3.reshape(1, -1))
    a4 = _make_agg(16)(m4, src, dst, z16)
    return _fin(a4, m4, dinv, W4, b4.reshape(1, -1))


# speedup vs baseline: 1.9265x; 1.2186x over previous
import functools

import jax
import jax.numpy as jnp
from jax import lax
from jax.experimental import pallas as pl
from jax.experimental.pallas import tpu as pltpu
from jax.experimental.pallas import tpu_sc as plsc

N = 10000
E = 320000
NC = 2
NS = 16
NW = NC * NS
CH = 80
EPW = E // NW
NCHUNK = EPW // CH
NPAD = 10240
RPW = NPAD // NS
DEGW = 16


@functools.cache
def _mesh():
    return plsc.VectorSubcoreMesh(
        core_axis_name="c", subcore_axis_name="s", num_cores=NC, num_subcores=NS)


@functools.cache
def _make_deg():
    @functools.partial(
        pl.kernel,
        out_type=jax.ShapeDtypeStruct((NC, NPAD, DEGW), jnp.float32),
        mesh=_mesh(),
        scratch_types=[
            pltpu.VMEM((NCHUNK, CH), jnp.int32),
            pltpu.VMEM((CH, DEGW), jnp.float32),
            pltpu.VMEM_SHARED((NPAD, DEGW), jnp.float32),
        ],
        compiler_params=pltpu.CompilerParams(use_tc_tiling_on_sc=False),
    )
    def _deg_kernel(dst_hbm, ones_hbm, zeros_hbm, out_hbm, dst_v, ones_v, acc):
        cid = lax.axis_index("c")
        sid = lax.axis_index("s")
        wid = sid * NC + cid
        rows = pl.ds(sid * RPW, RPW)
        pltpu.sync_copy(zeros_hbm.at[rows], acc.at[rows])
        pltpu.sync_copy(dst_hbm.at[wid], dst_v)
        pltpu.sync_copy(ones_hbm, ones_v)
        plsc.subcore_barrier()

        def body(j, carry):
            pltpu.sync_copy(ones_v, acc.at[dst_v.at[j]], add=True)
            return carry

        lax.fori_loop(0, NCHUNK, body, 0)
        plsc.subcore_barrier()
        pltpu.sync_copy(acc.at[rows], out_hbm.at[cid, rows])

    return _deg_kernel


@functools.cache
def _make_agg(d):
    @functools.partial(
        pl.kernel,
        out_type=jax.ShapeDtypeStruct((NC, NPAD, d), jnp.float32),
        mesh=_mesh(),
        scratch_types=[
            pltpu.VMEM((NCHUNK, CH), jnp.int32),
            pltpu.VMEM((NCHUNK, CH), jnp.int32),
            pltpu.VMEM((4, CH, d), jnp.float32),
            pltpu.VMEM_SHARED((NPAD, d), jnp.float32),
            [pltpu.SemaphoreType.DMA] * 4,
            [pltpu.SemaphoreType.DMA] * 4,
        ],
        compiler_params=pltpu.CompilerParams(use_tc_tiling_on_sc=False),
    )
    def k(m_hbm, src_hbm, dst_hbm, zeros_hbm, out_hbm, src_v, dst_v, gbuf, acc,
          gs, ss):
        cid = lax.axis_index("c")
        sid = lax.axis_index("s")
        wid = sid * NC + cid
        rows = pl.ds(sid * RPW, RPW)
        pltpu.sync_copy(zeros_hbm.at[rows], acc.at[rows])
        pltpu.sync_copy(src_hbm.at[wid], src_v)
        pltpu.sync_copy(dst_hbm.at[wid], dst_v)
        plsc.subcore_barrier()

        nmain = (NCHUNK - 1) // 4
        for k_ in range(4):
            pltpu.async_copy(m_hbm.at[src_v.at[k_]], gbuf.at[k_], gs[k_])

        def body(g, carry):
            j0 = 4 * g
            for k_ in range(4):
                j = j0 + k_
                pltpu.make_async_copy(m_hbm.at[src_v.at[j]], gbuf.at[k_],
                                      gs[k_]).wait()
                pltpu.async_copy(gbuf.at[k_], acc.at[dst_v.at[j]], ss[k_],
                                 add=True)
            for k_ in range(4):
                j = j0 + k_
                j2 = j + 4

                @pl.when(j2 < NCHUNK - 1)
                def _(k_=k_, j=j, j2=j2):
                    pltpu.make_async_copy(gbuf.at[k_], acc.at[dst_v.at[j]],
                                          ss[k_]).wait()
                    pltpu.async_copy(m_hbm.at[src_v.at[j2]], gbuf.at[k_],
                                     gs[k_])

            return carry

        lax.fori_loop(0, nmain, body, 0)
        for k_ in range(4):
            j = 4 * (nmain - 1) + k_
            pltpu.make_async_copy(gbuf.at[k_], acc.at[dst_v.at[j]],
                                  ss[k_]).wait()
        pltpu.async_copy(m_hbm.at[src_v.at[NCHUNK - 1]], gbuf.at[0], gs[0])
        pltpu.make_async_copy(m_hbm.at[src_v.at[NCHUNK - 1]], gbuf.at[0],
                              gs[0]).wait()
        pltpu.sync_copy(gbuf.at[0], acc.at[dst_v.at[NCHUNK - 1]], add=True)
        plsc.subcore_barrier()
        pltpu.sync_copy(acc.at[rows], out_hbm.at[cid, rows])

    return k


def _pre_body(x_ref, w_ref, degp_ref, m_ref, dinv_ref):
    deg = degp_ref[0][0:N, 0:1] + degp_ref[1][0:N, 0:1] + 1.0
    dinv = lax.rsqrt(deg)
    dinv_ref[...] = dinv
    m_ref[...] = jnp.dot(x_ref[...], w_ref[...],
                         preferred_element_type=jnp.float32) * dinv


_pre = pl.pallas_call(
    _pre_body,
    out_shape=[jax.ShapeDtypeStruct((N, 64), jnp.float32),
               jax.ShapeDtypeStruct((N, 1), jnp.float32)],
)


def _mid_body(a_ref, m_ref, dinv_ref, b_ref, g_ref, bt_ref, w_ref, o_ref):
    dinv = dinv_ref[...]
    t = (a_ref[0][0:N] + a_ref[1][0:N] + m_ref[...]) * dinv + b_ref[...]
    mu = jnp.mean(t, axis=0, keepdims=True)
    var = jnp.mean(jnp.square(t - mu), axis=0, keepdims=True)
    t = (t - mu) * lax.rsqrt(var + 1e-5) * g_ref[...] + bt_ref[...]
    t = jnp.maximum(t, 0.0)
    o_ref[...] = jnp.dot(t, w_ref[...], preferred_element_type=jnp.float32) * dinv


def _mid_nomat_body(a_ref, m_ref, dinv_ref, b_ref, g_ref, bt_ref, o_ref):
    dinv = dinv_ref[...]
    t = (a_ref[0][0:N] + a_ref[1][0:N] + m_ref[...]) * dinv + b_ref[...]
    mu = jnp.mean(t, axis=0, keepdims=True)
    var = jnp.mean(jnp.square(t - mu), axis=0, keepdims=True)
    t = (t - mu) * lax.rsqrt(var + 1e-5) * g_ref[...] + bt_ref[...]
    t = jnp.maximum(t, 0.0)
    o_ref[...] = t * dinv


def _fin_body(a_ref, m_ref, dinv_ref, w_ref, b_ref, o_ref):
    t = (a_ref[0][0:N] + a_ref[1][0:N] + m_ref[...]) * dinv_ref[...]
    h = jnp.dot(t, w_ref[...], preferred_element_type=jnp.float32) + b_ref[...]
    mx = jnp.max(h, axis=1, keepdims=True)
    lse = jnp.log(jnp.sum(jnp.exp(h - mx), axis=1, keepdims=True)) + mx
    o_ref[...] = h - lse


def _make_mid(dout):
    return pl.pallas_call(
        _mid_body, out_shape=jax.ShapeDtypeStruct((N, dout), jnp.float32))


_mid12 = _make_mid(32)
_mid23 = _make_mid(16)
_mid34 = pl.pallas_call(
    _mid_nomat_body, out_shape=jax.ShapeDtypeStruct((N, 16), jnp.float32))
_fin = pl.pallas_call(
    _fin_body, out_shape=jax.ShapeDtypeStruct((N, 40), jnp.float32))


def kernel(x, W1, b1, g1, bt1, W2, b2, g2, bt2, W3, b3, g3, bt3, W4, b4,
           edge_index):
    src = edge_index[0].reshape(NW, NCHUNK, CH)
    dst = edge_index[1].reshape(NW, NCHUNK, CH)
    ones = jnp.ones((CH, DEGW), jnp.float32)
    z_deg = jnp.zeros((NPAD, DEGW), jnp.float32)
    z64 = jnp.zeros((NPAD, 64), jnp.float32)
    z32 = jnp.zeros((NPAD, 32), jnp.float32)
    z16 = jnp.zeros((NPAD, 16), jnp.float32)

    degp = _make_deg()(dst, ones, z_deg)
    m1, dinv = _pre(x, W1, degp)
    a1 = _make_agg(64)(m1, src, dst, z64)
    m2 = _mid12(a1, m1, dinv, b1.reshape(1, -1), g1.reshape(1, -1),
                bt1.reshape(1, -1), W2)
    a2 = _make_agg(32)(m2, src, dst, z32)
    m3 = _mid23(a2, m2, dinv, b2.reshape(1, -1), g2.reshape(1, -1),
                bt2.reshape(1, -1), W3)
    a3 = _make_agg(16)(m3, src, dst, z16)
    m4 = _mid34(a3, m3, dinv, b3.reshape(1, -1), g3.reshape(1, -1),
                bt3.reshape(1, -1))
    a4 = _make_agg(16)(m4, src, dst, z16)
    return _fin(a4, m4, dinv, W4, b4.reshape(1, -1))


# speedup vs baseline: 2.0674x; 1.0732x over previous
import functools

import jax
import jax.numpy as jnp
from jax import lax
from jax.experimental import pallas as pl
from jax.experimental.pallas import tpu as pltpu
from jax.experimental.pallas import tpu_sc as plsc

N = 10000
E = 320000
NC = 2
NS = 16
NW = NC * NS
CH = 80
EPW = E // NW
NCHUNK = EPW // CH
RING = 5
NPAD = 10240
RPW = NPAD // NS
DEGW = 16


@functools.cache
def _mesh():
    return plsc.VectorSubcoreMesh(
        core_axis_name="c", subcore_axis_name="s", num_cores=NC, num_subcores=NS)


@functools.cache
def _make_deg():
    @functools.partial(
        pl.kernel,
        out_type=jax.ShapeDtypeStruct((NC, NPAD, DEGW), jnp.float32),
        mesh=_mesh(),
        scratch_types=[
            pltpu.VMEM((NCHUNK, CH), jnp.int32),
            pltpu.VMEM((CH, DEGW), jnp.float32),
            pltpu.VMEM_SHARED((NPAD, DEGW), jnp.float32),
            pltpu.SemaphoreType.DMA,
        ],
        compiler_params=pltpu.CompilerParams(use_tc_tiling_on_sc=False),
    )
    def _deg_kernel(dst_hbm, ones_hbm, zeros_hbm, out_hbm, dst_v, ones_v, acc,
                    sem):
        cid = lax.axis_index("c")
        sid = lax.axis_index("s")
        wid = sid * NC + cid
        rows = pl.ds(sid * RPW, RPW)
        pltpu.sync_copy(zeros_hbm.at[rows], acc.at[rows])
        pltpu.sync_copy(dst_hbm.at[wid], dst_v)
        pltpu.sync_copy(ones_hbm, ones_v)
        plsc.subcore_barrier()

        def fire(j, carry):
            pltpu.async_copy(ones_v, acc.at[dst_v.at[j]], sem, add=True)
            return carry

        lax.fori_loop(0, NCHUNK, fire, 0)

        def drain(j, carry):
            pltpu.make_async_copy(ones_v, acc.at[dst_v.at[j]], sem).wait()
            return carry

        lax.fori_loop(0, NCHUNK, drain, 0)
        plsc.subcore_barrier()
        pltpu.sync_copy(acc.at[rows], out_hbm.at[cid, rows])

    return _deg_kernel


@functools.cache
def _make_agg(d):
    @functools.partial(
        pl.kernel,
        out_type=jax.ShapeDtypeStruct((NC, NPAD, d), jnp.float32),
        mesh=_mesh(),
        scratch_types=[
            pltpu.VMEM((NCHUNK, CH), jnp.int32),
            pltpu.VMEM((NCHUNK, CH), jnp.int32),
            pltpu.VMEM((RING, CH, d), jnp.float32),
            pltpu.VMEM_SHARED((NPAD, d), jnp.float32),
            [pltpu.SemaphoreType.DMA] * RING,
            [pltpu.SemaphoreType.DMA] * RING,
        ],
        compiler_params=pltpu.CompilerParams(use_tc_tiling_on_sc=False),
    )
    def k(m_hbm, src_hbm, dst_hbm, zeros_hbm, out_hbm, src_v, dst_v, gbuf, acc,
          gs, ss):
        cid = lax.axis_index("c")
        sid = lax.axis_index("s")
        wid = sid * NC + cid
        rows = pl.ds(sid * RPW, RPW)
        pltpu.sync_copy(zeros_hbm.at[rows], acc.at[rows])
        pltpu.sync_copy(src_hbm.at[wid], src_v)
        pltpu.sync_copy(dst_hbm.at[wid], dst_v)
        plsc.subcore_barrier()

        for k_ in range(RING):
            pltpu.async_copy(m_hbm.at[src_v.at[k_]], gbuf.at[k_], gs[k_])

        def body(g, carry):
            j0 = RING * g
            for k_ in range(RING):
                j = j0 + k_
                pltpu.make_async_copy(m_hbm.at[src_v.at[j]], gbuf.at[k_],
                                      gs[k_]).wait()
                pltpu.async_copy(gbuf.at[k_], acc.at[dst_v.at[j]], ss[k_],
                                 add=True)
            for k_ in range(RING):
                j = j0 + k_
                j2 = j + RING

                @pl.when(j2 < NCHUNK)
                def _(k_=k_, j=j, j2=j2):
                    pltpu.make_async_copy(gbuf.at[k_], acc.at[dst_v.at[j]],
                                          ss[k_]).wait()
                    pltpu.async_copy(m_hbm.at[src_v.at[j2]], gbuf.at[k_],
                                     gs[k_])

            return carry

        lax.fori_loop(0, NCHUNK // RING, body, 0)
        for k_ in range(RING):
            j = NCHUNK - RING + k_
            pltpu.make_async_copy(gbuf.at[k_], acc.at[dst_v.at[j]],
                                  ss[k_]).wait()
        plsc.subcore_barrier()
        pltpu.sync_copy(acc.at[rows], out_hbm.at[cid, rows])

    return k


def _pre_body(x_ref, w_ref, degp_ref, m_ref, dinv_ref):
    deg = degp_ref[0][0:N, 0:1] + degp_ref[1][0:N, 0:1] + 1.0
    dinv = lax.rsqrt(deg)
    dinv_ref[...] = dinv
    m_ref[...] = jnp.dot(x_ref[...], w_ref[...],
                         preferred_element_type=jnp.float32) * dinv


_pre = pl.pallas_call(
    _pre_body,
    out_shape=[jax.ShapeDtypeStruct((N, 64), jnp.float32),
               jax.ShapeDtypeStruct((N, 1), jnp.float32)],
)


def _mid_body(a_ref, m_ref, dinv_ref, b_ref, g_ref, bt_ref, w_ref, o_ref):
    dinv = dinv_ref[...]
    t = (a_ref[0][0:N] + a_ref[1][0:N] + m_ref[...]) * dinv + b_ref[...]
    mu = jnp.mean(t, axis=0, keepdims=True)
    var = jnp.mean(jnp.square(t - mu), axis=0, keepdims=True)
    t = (t - mu) * lax.rsqrt(var + 1e-5) * g_ref[...] + bt_ref[...]
    t = jnp.maximum(t, 0.0)
    o_ref[...] = jnp.dot(t, w_ref[...], preferred_element_type=jnp.float32) * dinv


def _mid_nomat_body(a_ref, m_ref, dinv_ref, b_ref, g_ref, bt_ref, o_ref):
    dinv = dinv_ref[...]
    t = (a_ref[0][0:N] + a_ref[1][0:N] + m_ref[...]) * dinv + b_ref[...]
    mu = jnp.mean(t, axis=0, keepdims=True)
    var = jnp.mean(jnp.square(t - mu), axis=0, keepdims=True)
    t = (t - mu) * lax.rsqrt(var + 1e-5) * g_ref[...] + bt_ref[...]
    t = jnp.maximum(t, 0.0)
    o_ref[...] = t * dinv


def _fin_body(a_ref, m_ref, dinv_ref, w_ref, b_ref, o_ref):
    t = (a_ref[0][0:N] + a_ref[1][0:N] + m_ref[...]) * dinv_ref[...]
    h = jnp.dot(t, w_ref[...], preferred_element_type=jnp.float32) + b_ref[...]
    mx = jnp.max(h, axis=1, keepdims=True)
    lse = jnp.log(jnp.sum(jnp.exp(h - mx), axis=1, keepdims=True)) + mx
    o_ref[...] = h - lse


def _make_mid(dout):
    return pl.pallas_call(
        _mid_body, out_shape=jax.ShapeDtypeStruct((N, dout), jnp.float32))


_mid12 = _make_mid(32)
_mid23 = _make_mid(16)
_mid34 = pl.pallas_call(
    _mid_nomat_body, out_shape=jax.ShapeDtypeStruct((N, 16), jnp.float32))
_fin = pl.pallas_call(
    _fin_body, out_shape=jax.ShapeDtypeStruct((N, 40), jnp.float32))


def kernel(x, W1, b1, g1, bt1, W2, b2, g2, bt2, W3, b3, g3, bt3, W4, b4,
           edge_index):
    src = edge_index[0].reshape(NW, NCHUNK, CH)
    dst = edge_index[1].reshape(NW, NCHUNK, CH)
    ones = jnp.ones((CH, DEGW), jnp.float32)
    z_deg = jnp.zeros((NPAD, DEGW), jnp.float32)
    z64 = jnp.zeros((NPAD, 64), jnp.float32)
    z32 = jnp.zeros((NPAD, 32), jnp.float32)
    z16 = jnp.zeros((NPAD, 16), jnp.float32)

    degp = _make_deg()(dst, ones, z_deg)
    m1, dinv = _pre(x, W1, degp)
    a1 = _make_agg(64)(m1, src, dst, z64)
    m2 = _mid12(a1, m1, dinv, b1.reshape(1, -1), g1.reshape(1, -1),
                bt1.reshape(1, -1), W2)
    a2 = _make_agg(32)(m2, src, dst, z32)
    m3 = _mid23(a2, m2, dinv, b2.reshape(1, -1), g2.reshape(1, -1),
                bt2.reshape(1, -1), W3)
    a3 = _make_agg(16)(m3, src, dst, z16)
    m4 = _mid34(a3, m3, dinv, b3.reshape(1, -1), g3.reshape(1, -1),
                bt3.reshape(1, -1))
    a4 = _make_agg(16)(m4, src, dst, z16)
    return _fin(a4, m4, dinv, W4, b4.reshape(1, -1))


# speedup vs baseline: 2.2165x; 1.0721x over previous
import functools

import jax
import jax.numpy as jnp
from jax import lax
from jax.experimental import pallas as pl
from jax.experimental.pallas import tpu as pltpu
from jax.experimental.pallas import tpu_sc as plsc

N = 10000
E = 320000
NC = 2
NS = 16
NW = NC * NS
CH = 200
EPW = E // NW
NCHUNK = EPW // CH
RING = 5
NPAD = 10240
RPW = NPAD // NS
DEGW = 16


@functools.cache
def _mesh():
    return plsc.VectorSubcoreMesh(
        core_axis_name="c", subcore_axis_name="s", num_cores=NC, num_subcores=NS)


@functools.cache
def _make_deg():
    @functools.partial(
        pl.kernel,
        out_type=jax.ShapeDtypeStruct((NC, NPAD, DEGW), jnp.float32),
        mesh=_mesh(),
        scratch_types=[
            pltpu.VMEM((NCHUNK, CH), jnp.int32),
            pltpu.VMEM((CH, DEGW), jnp.float32),
            pltpu.VMEM_SHARED((NPAD, DEGW), jnp.float32),
            pltpu.SemaphoreType.DMA,
        ],
        compiler_params=pltpu.CompilerParams(use_tc_tiling_on_sc=False),
    )
    def _deg_kernel(dst_hbm, ones_hbm, zeros_hbm, out_hbm, dst_v, ones_v, acc,
                    sem):
        cid = lax.axis_index("c")
        sid = lax.axis_index("s")
        wid = sid * NC + cid
        rows = pl.ds(sid * RPW, RPW)
        pltpu.sync_copy(zeros_hbm.at[rows], acc.at[rows])
        pltpu.sync_copy(dst_hbm.at[wid], dst_v)
        pltpu.sync_copy(ones_hbm, ones_v)
        plsc.subcore_barrier()

        def fire(j, carry):
            pltpu.async_copy(ones_v, acc.at[dst_v.at[j]], sem, add=True)
            return carry

        lax.fori_loop(0, NCHUNK, fire, 0)

        def drain(j, carry):
            pltpu.make_async_copy(ones_v, acc.at[dst_v.at[j]], sem).wait()
            return carry

        lax.fori_loop(0, NCHUNK, drain, 0)
        plsc.subcore_barrier()
        pltpu.sync_copy(acc.at[rows], out_hbm.at[cid, rows])

    return _deg_kernel


@functools.cache
def _make_agg(d):
    @functools.partial(
        pl.kernel,
        out_type=jax.ShapeDtypeStruct((NC, NPAD, d), jnp.float32),
        mesh=_mesh(),
        scratch_types=[
            pltpu.VMEM((NCHUNK, CH), jnp.int32),
            pltpu.VMEM((NCHUNK, CH), jnp.int32),
            pltpu.VMEM((RING, CH, d), jnp.float32),
            pltpu.VMEM_SHARED((NPAD, d), jnp.float32),
            [pltpu.SemaphoreType.DMA] * RING,
            [pltpu.SemaphoreType.DMA] * RING,
        ],
        compiler_params=pltpu.CompilerParams(use_tc_tiling_on_sc=False),
    )
    def k(m_hbm, src_hbm, dst_hbm, zeros_hbm, out_hbm, src_v, dst_v, gbuf, acc,
          gs, ss):
        cid = lax.axis_index("c")
        sid = lax.axis_index("s")
        wid = sid * NC + cid
        rows = pl.ds(sid * RPW, RPW)
        pltpu.sync_copy(zeros_hbm.at[rows], acc.at[rows])
        pltpu.sync_copy(src_hbm.at[wid], src_v)
        pltpu.sync_copy(dst_hbm.at[wid], dst_v)
        plsc.subcore_barrier()

        for k_ in range(RING):
            pltpu.async_copy(m_hbm.at[src_v.at[k_]], gbuf.at[k_], gs[k_])

        def body(g, carry):
            j0 = RING * g
            for k_ in range(RING):
                j = j0 + k_
                pltpu.make_async_copy(m_hbm.at[src_v.at[j]], gbuf.at[k_],
                                      gs[k_]).wait()
                pltpu.async_copy(gbuf.at[k_], acc.at[dst_v.at[j]], ss[k_],
                                 add=True)
            for k_ in range(RING):
                j = j0 + k_
                j2 = j + RING

                @pl.when(j2 < NCHUNK)
                def _(k_=k_, j=j, j2=j2):
                    pltpu.make_async_copy(gbuf.at[k_], acc.at[dst_v.at[j]],
                                          ss[k_]).wait()
                    pltpu.async_copy(m_hbm.at[src_v.at[j2]], gbuf.at[k_],
                                     gs[k_])

            return carry

        lax.fori_loop(0, NCHUNK // RING, body, 0)
        for k_ in range(RING):
            j = NCHUNK - RING + k_
            pltpu.make_async_copy(gbuf.at[k_], acc.at[dst_v.at[j]],
                                  ss[k_]).wait()
        plsc.subcore_barrier()
        pltpu.sync_copy(acc.at[rows], out_hbm.at[cid, rows])

    return k


def _pre_body(x_ref, w_ref, degp_ref, m_ref, dinv_ref):
    deg = degp_ref[0][0:N, 0:1] + degp_ref[1][0:N, 0:1] + 1.0
    dinv = lax.rsqrt(deg)
    dinv_ref[...] = dinv
    m_ref[...] = jnp.dot(x_ref[...], w_ref[...],
                         preferred_element_type=jnp.float32) * dinv


_pre = pl.pallas_call(
    _pre_body,
    out_shape=[jax.ShapeDtypeStruct((N, 64), jnp.float32),
               jax.ShapeDtypeStruct((N, 1), jnp.float32)],
)


def _mid_body(a_ref, m_ref, dinv_ref, b_ref, g_ref, bt_ref, w_ref, o_ref):
    dinv = dinv_ref[...]
    t = (a_ref[0][0:N] + a_ref[1][0:N] + m_ref[...]) * dinv + b_ref[...]
    mu = jnp.mean(t, axis=0, keepdims=True)
    var = jnp.mean(jnp.square(t - mu), axis=0, keepdims=True)
    t = (t - mu) * lax.rsqrt(var + 1e-5) * g_ref[...] + bt_ref[...]
    t = jnp.maximum(t, 0.0)
    o_ref[...] = jnp.dot(t, w_ref[...], preferred_element_type=jnp.float32) * dinv


def _mid_nomat_body(a_ref, m_ref, dinv_ref, b_ref, g_ref, bt_ref, o_ref):
    dinv = dinv_ref[...]
    t = (a_ref[0][0:N] + a_ref[1][0:N] + m_ref[...]) * dinv + b_ref[...]
    mu = jnp.mean(t, axis=0, keepdims=True)
    var = jnp.mean(jnp.square(t - mu), axis=0, keepdims=True)
    t = (t - mu) * lax.rsqrt(var + 1e-5) * g_ref[...] + bt_ref[...]
    t = jnp.maximum(t, 0.0)
    o_ref[...] = t * dinv


def _fin_body(a_ref, m_ref, dinv_ref, w_ref, b_ref, o_ref):
    t = (a_ref[0][0:N] + a_ref[1][0:N] + m_ref[...]) * dinv_ref[...]
    h = jnp.dot(t, w_ref[...], preferred_element_type=jnp.float32) + b_ref[...]
    mx = jnp.max(h, axis=1, keepdims=True)
    lse = jnp.log(jnp.sum(jnp.exp(h - mx), axis=1, keepdims=True)) + mx
    o_ref[...] = h - lse


def _make_mid(dout):
    return pl.pallas_call(
        _mid_body, out_shape=jax.ShapeDtypeStruct((N, dout), jnp.float32))


_mid12 = _make_mid(32)
_mid23 = _make_mid(16)
_mid34 = pl.pallas_call(
    _mid_nomat_body, out_shape=jax.ShapeDtypeStruct((N, 16), jnp.float32))
_fin = pl.pallas_call(
    _fin_body, out_shape=jax.ShapeDtypeStruct((N, 40), jnp.float32))


def kernel(x, W1, b1, g1, bt1, W2, b2, g2, bt2, W3, b3, g3, bt3, W4, b4,
           edge_index):
    src = edge_index[0].reshape(NW, NCHUNK, CH)
    dst = edge_index[1].reshape(NW, NCHUNK, CH)
    ones = jnp.ones((CH, DEGW), jnp.float32)
    z_deg = jnp.zeros((NPAD, DEGW), jnp.float32)
    z64 = jnp.zeros((NPAD, 64), jnp.float32)
    z32 = jnp.zeros((NPAD, 32), jnp.float32)
    z16 = jnp.zeros((NPAD, 16), jnp.float32)

    degp = _make_deg()(dst, ones, z_deg)
    m1, dinv = _pre(x, W1, degp)
    a1 = _make_agg(64)(m1, src, dst, z64)
    m2 = _mid12(a1, m1, dinv, b1.reshape(1, -1), g1.reshape(1, -1),
                bt1.reshape(1, -1), W2)
    a2 = _make_agg(32)(m2, src, dst, z32)
    m3 = _mid23(a2, m2, dinv, b2.reshape(1, -1), g2.reshape(1, -1),
                bt2.reshape(1, -1), W3)
    a3 = _make_agg(16)(m3, src, dst, z16)
    m4 = _mid34(a3, m3, dinv, b3.reshape(1, -1), g3.reshape(1, -1),
                bt3.reshape(1, -1))
    a4 = _make_agg(16)(m4, src, dst, z16)
    return _fin(a4, m4, dinv, W4, b4.reshape(1, -1))
